# baseline (speedup 1.0000x reference)
import jax
import jax.numpy as jnp
from jax import lax
from jax.experimental import pallas as pl
from jax.experimental.pallas import tpu as pltpu
from jax.experimental.pallas import tpu_sc as plsc

N = 10000
E = 160000
D = 128
DE = D + 16
NT = 16
EPT = E // NT
CH = 80
NCH = EPT // CH
STR = 624
F32 = jnp.float32
I32 = jnp.int32


def _sc_body(hs, ald, packed, acc,
             packed_v, sidx0, sidx1, didx0, didx1,
             rows0, rows1, adc0, adc1, px0, px1,
             gsem0, gsem1, ssem0, ssem1,
             accs):
    c = lax.axis_index("c")
    s = lax.axis_index("s")
    sidx = [sidx0, sidx1]
    didx = [didx0, didx1]
    rows = [rows0, rows1]
    adc = [adc0, adc1]
    px = [px0, px1]
    gsem = [gsem0, gsem1]
    ssem = [ssem0, ssem1]

    def zrow_step(r, carry):
        for j in range(DE // 16):
            rows0[r, pl.ds(j * 16, 16)] = jnp.zeros((16,), F32)
        return carry
    lax.fori_loop(0, CH, zrow_step, 0, unroll=True)

    for k in range(7):
        pltpu.sync_copy(rows0, accs.at[pl.ds(s * STR + k * CH, CH), :])
    pltpu.sync_copy(rows0.at[pl.ds(0, 64), :],
                    accs.at[pl.ds(s * STR + 560, 64), :])

    @pl.when(s == NT - 1)
    def _():
        pltpu.sync_copy(rows0.at[pl.ds(0, 16), :],
                        accs.at[pl.ds(NT * STR, 16), :])

    pltpu.sync_copy(packed.at[c, s], packed_v)
    plsc.subcore_barrier()

    rowid = jnp.arange(16, dtype=I32)
    zero16 = jnp.zeros((16,), I32)
    col_d = jnp.full((16,), D, I32)

    def issue(rr, b):
        @plsc.parallel_loop(0, CH // 16)
        def _(j):
            p = packed_v[rr, pl.ds(j * 16, 16)]
            sidx[b][pl.ds(j * 16, 16)] = p & 0xFFFF
            didx[b][pl.ds(j * 16, 16)] = p >> 16
        pltpu.async_copy(ald.at[c].at[didx[b]], adc[b], gsem[b])
        pltpu.async_copy(hs.at[c].at[sidx[b]], rows[b], gsem[b])

    def drain_scatter(b):
        pltpu.make_async_copy(rows[b], accs.at[didx[b]], ssem[b]).wait()

    def consume(b, async_scatter):
        pltpu.make_async_copy(ald.at[c].at[didx[b]], adc[b], gsem[b]).wait()
        pltpu.make_async_copy(hs.at[c].at[sidx[b]], rows[b], gsem[b]).wait()
        for j in range(CH // 16):
            ridx = rowid + (j * 16)
            a = (plsc.load_gather(rows[b], [ridx, col_d])
                 + plsc.load_gather(adc[b], [ridx, zero16]))
            a = jnp.where(a > 0, a, 0.2 * a)
            px[b][pl.ds(j * 16, 16)] = jnp.exp(a)

        @plsc.parallel_loop(0, CH, unroll=4)
        def _(e):
            w = plsc.load_gather(px[b], [jnp.full((16,), e, I32)])
            for j in range(D // 16):
                rows[b][e, pl.ds(j * 16, 16)] = (
                    rows[b][e, pl.ds(j * 16, 16)] * w)
            rows[b][e, pl.ds(D, 16)] = w

        if async_scatter:
            pltpu.async_copy(rows[b], accs.at[didx[b]], ssem[b], add=True)
        else:
            pltpu.sync_copy(rows[b], accs.at[didx[b]], add=True)

    issue(0, 0)

    def step2(i, carry):
        r = i * 2
        @pl.when(i > 0)
        def _():
            drain_scatter(1)
        issue(r + 1, 1)
        consume(0, True)
        drain_scatter(0)

        @pl.when(r + 2 < NCH)
        def _():
            issue(r + 2, 0)
        consume(1, True)
        return carry
    lax.fori_loop(0, NCH // 2, step2, 0)
    drain_scatter(1)
    consume(0, False)
    plsc.subcore_barrier()

    pltpu.sync_copy(accs.at[pl.ds(s * STR, STR), :],
                    acc.at[c].at[pl.ds(s * STR, STR), :])

    @pl.when(s == NT - 1)
    def _():
        pltpu.sync_copy(accs.at[pl.ds(NT * STR, 16), :],
                        acc.at[c].at[pl.ds(NT * STR, 16), :])


_sc_gat = pl.kernel(
    _sc_body,
    out_type=jax.ShapeDtypeStruct((2, N, DE), F32),
    mesh=plsc.VectorSubcoreMesh(core_axis_name="c", subcore_axis_name="s"),
    compiler_params=pltpu.CompilerParams(needs_layout_passes=False,
                                         use_tc_tiling_on_sc=False),
    scratch_types=[
        pltpu.VMEM((NCH, CH), I32),
        pltpu.VMEM((CH,), I32),
        pltpu.VMEM((CH,), I32),
        pltpu.VMEM((CH,), I32),
        pltpu.VMEM((CH,), I32),
        pltpu.VMEM((CH, DE), F32),
        pltpu.VMEM((CH, DE), F32),
        pltpu.VMEM((CH, 16), F32),
        pltpu.VMEM((CH, 16), F32),
        pltpu.VMEM((CH,), F32),
        pltpu.VMEM((CH,), F32),
        pltpu.SemaphoreType.DMA,
        pltpu.SemaphoreType.DMA,
        pltpu.SemaphoreType.DMA,
        pltpu.SemaphoreType.DMA,
        pltpu.VMEM_SHARED((N, DE), F32),
    ],
)


BLK = 1000
GRID = N // BLK


def _elu(x):
    return jnp.where(x > 0, x, jnp.exp(jnp.minimum(x, 0.0)) - 1.0)


def _bcast16(v):
    return jnp.broadcast_to(v, (v.shape[0], 16))


def _combine(acc_rel, b):
    return _elu(acc_rel[:, :D] / (acc_rel[:, D:D + 1] + 1e-16) + b)


def _proj_common(hu, hi, W_iu, as_iu, ad_iu, W_ui, as_ui, ad_ui,
                 hs_ref, ald_ref):
    s_iu = jnp.dot(hi, W_iu, preferred_element_type=F32)
    d_iu = jnp.dot(hu, W_iu, preferred_element_type=F32)
    s_ui = jnp.dot(hu, W_ui, preferred_element_type=F32)
    d_ui = jnp.dot(hi, W_ui, preferred_element_type=F32)
    als_iu = _bcast16(jnp.sum(s_iu * as_iu, axis=1, keepdims=True))
    als_ui = _bcast16(jnp.sum(s_ui * as_ui, axis=1, keepdims=True))
    hs_ref[0] = jnp.concatenate([s_iu, als_iu], axis=1)
    hs_ref[1] = jnp.concatenate([s_ui, als_ui], axis=1)
    ald_ref[0] = _bcast16(jnp.sum(d_iu * ad_iu, axis=1, keepdims=True))
    ald_ref[1] = _bcast16(jnp.sum(d_ui * ad_ui, axis=1, keepdims=True))


def _proj0_body(xu_ref, xi_ref, Wp_ref, bp_ref,
                W_iu_ref, as_iu_ref, ad_iu_ref,
                W_ui_ref, as_ui_ref, ad_ui_ref,
                hs_ref, ald_ref):
    Wp = Wp_ref[...]
    hu = jnp.dot(xu_ref[...], Wp, preferred_element_type=F32) + bp_ref[...]
    hi = jnp.dot(xi_ref[...], Wp, preferred_element_type=F32) + bp_ref[...]
    _proj_common(hu, hi, W_iu_ref[...], as_iu_ref[...], ad_iu_ref[...],
                 W_ui_ref[...], as_ui_ref[...], ad_ui_ref[...],
                 hs_ref, ald_ref)


def _proj1_body(acc_ref, biu_ref, bui_ref,
                W_iu_ref, as_iu_ref, ad_iu_ref,
                W_ui_ref, as_ui_ref, ad_ui_ref,
                hs_ref, ald_ref):
    hu = _combine(acc_ref[0], biu_ref[...])
    hi = _combine(acc_ref[1], bui_ref[...])
    _proj_common(hu, hi, W_iu_ref[...], as_iu_ref[...], ad_iu_ref[...],
                 W_ui_ref[...], as_ui_ref[...], ad_ui_ref[...],
                 hs_ref, ald_ref)


def _final_body(acc_ref, biu_ref, bui_ref, Wo_ref, bo_ref,
                outu_ref, hi2_ref):
    hu2 = _combine(acc_ref[0], biu_ref[...])
    hi2 = _combine(acc_ref[1], bui_ref[...])
    outu_ref[...] = (jnp.dot(hu2, Wo_ref[...], preferred_element_type=F32)
                     + bo_ref[...])
    hi2_ref[...] = hi2


def _row_spec(width):
    return pl.BlockSpec((BLK, width), lambda i: (i, 0))


def _st_spec(width):
    return pl.BlockSpec((2, BLK, width), lambda i: (0, i, 0))


def _full_spec(r, cdim):
    return pl.BlockSpec((r, cdim), lambda i: (0, 0))


_proj0 = pl.pallas_call(
    _proj0_body,
    grid=(GRID,),
    in_specs=[_row_spec(D), _row_spec(D), _full_spec(D, D), _full_spec(1, D),
              _full_spec(D, D), _full_spec(1, D), _full_spec(1, D),
              _full_spec(D, D), _full_spec(1, D), _full_spec(1, D)],
    out_specs=[_st_spec(DE), _st_spec(16)],
    out_shape=[jax.ShapeDtypeStruct((2, N, DE), F32),
               jax.ShapeDtypeStruct((2, N, 16), F32)],
)

_proj1 = pl.pallas_call(
    _proj1_body,
    grid=(GRID,),
    in_specs=[_st_spec(DE), _full_spec(1, D), _full_spec(1, D),
              _full_spec(D, D), _full_spec(1, D), _full_spec(1, D),
              _full_spec(D, D), _full_spec(1, D), _full_spec(1, D)],
    out_specs=[_st_spec(DE), _st_spec(16)],
    out_shape=[jax.ShapeDtypeStruct((2, N, DE), F32),
               jax.ShapeDtypeStruct((2, N, 16), F32)],
)

_final = pl.pallas_call(
    _final_body,
    grid=(GRID,),
    in_specs=[_st_spec(DE), _full_spec(1, D), _full_spec(1, D),
              _full_spec(D, 64), _full_spec(1, 64)],
    out_specs=[_row_spec(64), _row_spec(D)],
    out_shape=[jax.ShapeDtypeStruct((N, 64), F32),
               jax.ShapeDtypeStruct((N, D), F32)],
)


def kernel(x_user, x_item, edge_index_ui, edge_index_iu, Wp, bp,
           W_ui0, as_ui0, ad_ui0, b_ui0, W_iu0, as_iu0, ad_iu0, b_iu0,
           W_ui1, as_ui1, ad_ui1, b_ui1, W_iu1, as_iu1, ad_iu1, b_iu1,
           Wo, bo):
    ei_ui = edge_index_ui.astype(I32)
    ei_iu = edge_index_iu.astype(I32)
    packed = jnp.stack([
        (ei_iu[0] | (ei_iu[1] << 16)).reshape(NT, NCH, CH),
        (ei_ui[0] | (ei_ui[1] << 16)).reshape(NT, NCH, CH)])
    bp2 = bp.reshape(1, D)

    hs0, ald0 = _proj0(x_user, x_item, Wp, bp2,
                       W_iu0, as_iu0, ad_iu0,
                       W_ui0, as_ui0, ad_ui0)
    acc0 = _sc_gat(hs0, ald0, packed)

    hs1, ald1 = _proj1(acc0, b_iu0.reshape(1, D), b_ui0.reshape(1, D),
                       W_iu1, as_iu1, ad_iu1,
                       W_ui1, as_ui1, ad_ui1)
    acc1 = _sc_gat(hs1, ald1, packed)

    out_user, hi2 = _final(acc1, b_iu1.reshape(1, D), b_ui1.reshape(1, D),
                           Wo, bo.reshape(1, 64))
    return (out_user, hi2)


# speedup vs baseline: 1.1490x; 1.1490x over previous
import jax
import jax.numpy as jnp
from jax import lax
from jax.experimental import pallas as pl
from jax.experimental.pallas import tpu as pltpu
from jax.experimental.pallas import tpu_sc as plsc

N = 10000
E = 160000
D = 128
NT = 16
EPT = E // NT
CH = 80
NCH = EPT // CH
STR = 624
F32 = jnp.float32
I32 = jnp.int32


def _sc_body(hs, als, ald, packed, acc,
             packed_v, sidx0, sidx1, didx0, didx1,
             rows0, rows1, alc0, alc1, adc0, adc1, px0, px1, denr0, denr1,
             gsem0, gsem1, ssem0, ssem1,
             accs, dens):
    c = lax.axis_index("c")
    s = lax.axis_index("s")
    sidx = [sidx0, sidx1]
    didx = [didx0, didx1]
    rows = [rows0, rows1]
    alc = [alc0, alc1]
    adc = [adc0, adc1]
    px = [px0, px1]
    denr = [denr0, denr1]
    gsem = [gsem0, gsem1]
    ssem = [ssem0, ssem1]

    def zrow_step(r, carry):
        for j in range(D // 16):
            rows0[r, pl.ds(j * 16, 16)] = jnp.zeros((16,), F32)
        denr0[r, :] = jnp.zeros((16,), F32)
        return carry
    lax.fori_loop(0, CH, zrow_step, 0, unroll=True)

    for k in range(7):
        base = s * STR + k * CH
        pltpu.sync_copy(rows0, accs.at[pl.ds(base, CH), :])
        pltpu.sync_copy(denr0, dens.at[pl.ds(base, CH), :])
    pltpu.sync_copy(rows0.at[pl.ds(0, 64), :],
                    accs.at[pl.ds(s * STR + 560, 64), :])
    pltpu.sync_copy(denr0.at[pl.ds(0, 64), :],
                    dens.at[pl.ds(s * STR + 560, 64), :])

    @pl.when(s == NT - 1)
    def _():
        pltpu.sync_copy(rows0.at[pl.ds(0, 16), :],
                        accs.at[pl.ds(NT * STR, 16), :])
        pltpu.sync_copy(denr0.at[pl.ds(0, 16), :],
                        dens.at[pl.ds(NT * STR, 16), :])

    pltpu.sync_copy(packed.at[c, s], packed_v)
    plsc.subcore_barrier()

    rowid = jnp.arange(16, dtype=I32)
    zero16 = jnp.zeros((16,), I32)

    def issue(rr, b):
        @plsc.parallel_loop(0, CH // 16)
        def _(j):
            p = packed_v[rr, pl.ds(j * 16, 16)]
            sidx[b][pl.ds(j * 16, 16)] = p & 0xFFFF
            didx[b][pl.ds(j * 16, 16)] = p >> 16
        pltpu.async_copy(als.at[c].at[sidx[b]], alc[b], gsem[b])
        pltpu.async_copy(ald.at[c].at[didx[b]], adc[b], gsem[b])
        pltpu.async_copy(hs.at[c].at[sidx[b]], rows[b], gsem[b])

    def drain_scatter(b):
        pltpu.make_async_copy(rows[b], accs.at[didx[b]], ssem[b]).wait()
        pltpu.make_async_copy(denr[b], dens.at[didx[b]], ssem[b]).wait()

    def consume(b, async_scatter):
        pltpu.make_async_copy(als.at[c].at[sidx[b]], alc[b], gsem[b]).wait()
        pltpu.make_async_copy(ald.at[c].at[didx[b]], adc[b], gsem[b]).wait()
        pltpu.make_async_copy(hs.at[c].at[sidx[b]], rows[b], gsem[b]).wait()
        for j in range(CH // 16):
            ridx = rowid + (j * 16)
            a = (plsc.load_gather(alc[b], [ridx, zero16])
                 + plsc.load_gather(adc[b], [ridx, zero16]))
            a = jnp.where(a > 0, a, 0.2 * a)
            px[b][pl.ds(j * 16, 16)] = jnp.exp(a)

        @plsc.parallel_loop(0, CH, unroll=8)
        def _(e):
            w = plsc.load_gather(px[b], [jnp.full((16,), e, I32)])
            for j in range(D // 16):
                rows[b][e, pl.ds(j * 16, 16)] = (
                    rows[b][e, pl.ds(j * 16, 16)] * w)
            denr[b][e, :] = w

        if async_scatter:
            pltpu.async_copy(rows[b], accs.at[didx[b]], ssem[b], add=True)
            pltpu.async_copy(denr[b], dens.at[didx[b]], ssem[b], add=True)
        else:
            pltpu.sync_copy(rows[b], accs.at[didx[b]], add=True)
            pltpu.sync_copy(denr[b], dens.at[didx[b]], add=True)

    issue(0, 0)

    def step2(i, carry):
        r = i * 2
        @pl.when(i > 0)
        def _():
            drain_scatter(1)
        issue(r + 1, 1)
        consume(0, True)
        drain_scatter(0)

        @pl.when(r + 2 < NCH)
        def _():
            issue(r + 2, 0)
        consume(1, True)
        return carry
    lax.fori_loop(0, NCH // 2, step2, 0)
    drain_scatter(1)
    consume(0, False)
    plsc.subcore_barrier()

    def fl_nrows(k):
        return 64 if k == 7 else CH

    def fl_base(k):
        return s * STR + k * CH

    def fl_in(k, b):
        nr = fl_nrows(k)
        pltpu.async_copy(accs.at[pl.ds(fl_base(k), nr), :],
                         rows[b].at[pl.ds(0, nr), :], gsem[b])
        pltpu.async_copy(dens.at[pl.ds(fl_base(k), nr), :],
                         denr[b].at[pl.ds(0, nr), :], gsem[b])

    def fl_in_wait(k, b):
        nr = fl_nrows(k)
        pltpu.make_async_copy(accs.at[pl.ds(fl_base(k), nr), :],
                              rows[b].at[pl.ds(0, nr), :], gsem[b]).wait()
        pltpu.make_async_copy(dens.at[pl.ds(fl_base(k), nr), :],
                              denr[b].at[pl.ds(0, nr), :], gsem[b]).wait()

    def fl_norm(k, b):
        @plsc.parallel_loop(0, fl_nrows(k))
        def _(rr):
            inv = 1.0 / (denr[b][rr, :] + 1e-16)
            for j in range(D // 16):
                rows[b][rr, pl.ds(j * 16, 16)] = (
                    rows[b][rr, pl.ds(j * 16, 16)] * inv)

    def fl_out(k, b):
        nr = fl_nrows(k)
        pltpu.async_copy(rows[b].at[pl.ds(0, nr), :],
                         acc.at[c].at[pl.ds(fl_base(k), nr), :], ssem[b])

    def fl_out_wait(k, b):
        nr = fl_nrows(k)
        pltpu.make_async_copy(rows[b].at[pl.ds(0, nr), :],
                              acc.at[c].at[pl.ds(fl_base(k), nr), :],
                              ssem[b]).wait()

    fl_in(0, 0)
    for k in range(8):
        b = k & 1
        if k >= 1:
            fl_out_wait(k - 1, 1 - b)
        if k + 1 < 8:
            fl_in(k + 1, 1 - b)
        fl_in_wait(k, b)
        fl_norm(k, b)
        fl_out(k, b)
    fl_out_wait(7, 1)

    @pl.when(s == NT - 1)
    def _():
        pltpu.sync_copy(accs.at[pl.ds(NT * STR, 16), :],
                        rows0.at[pl.ds(0, 16), :])
        pltpu.sync_copy(dens.at[pl.ds(NT * STR, 16), :],
                        denr0.at[pl.ds(0, 16), :])

        @plsc.parallel_loop(0, 16)
        def _(rr):
            inv = 1.0 / (denr0[rr, :] + 1e-16)
            for j in range(D // 16):
                rows0[rr, pl.ds(j * 16, 16)] = (
                    rows0[rr, pl.ds(j * 16, 16)] * inv)
        pltpu.sync_copy(rows0.at[pl.ds(0, 16), :],
                        acc.at[c].at[pl.ds(NT * STR, 16), :])


_sc_gat = pl.kernel(
    _sc_body,
    out_type=jax.ShapeDtypeStruct((2, N, D), F32),
    mesh=plsc.VectorSubcoreMesh(core_axis_name="c", subcore_axis_name="s"),
    compiler_params=pltpu.CompilerParams(needs_layout_passes=False,
                                         use_tc_tiling_on_sc=False),
    scratch_types=[
        pltpu.VMEM((NCH, CH), I32),
        pltpu.VMEM((CH,), I32),
        pltpu.VMEM((CH,), I32),
        pltpu.VMEM((CH,), I32),
        pltpu.VMEM((CH,), I32),
        pltpu.VMEM((CH, D), F32),
        pltpu.VMEM((CH, D), F32),
        pltpu.VMEM((CH, 16), F32),
        pltpu.VMEM((CH, 16), F32),
        pltpu.VMEM((CH, 16), F32),
        pltpu.VMEM((CH, 16), F32),
        pltpu.VMEM((CH,), F32),
        pltpu.VMEM((CH,), F32),
        pltpu.VMEM((CH, 16), F32),
        pltpu.VMEM((CH, 16), F32),
        pltpu.SemaphoreType.DMA,
        pltpu.SemaphoreType.DMA,
        pltpu.SemaphoreType.DMA,
        pltpu.SemaphoreType.DMA,
        pltpu.VMEM_SHARED((N, D), F32),
        pltpu.VMEM_SHARED((N, 16), F32),
    ],
)


BLK = 1000
GRID = N // BLK


def _elu(x):
    return jnp.where(x > 0, x, jnp.exp(jnp.minimum(x, 0.0)) - 1.0)


def _bcast16(v):
    return jnp.broadcast_to(v, (v.shape[0], 16))


def _proj_common(hu, hi, W_iu, as_iu, ad_iu, W_ui, as_ui, ad_ui,
                 hs_ref, als_ref, ald_ref):
    s_iu = jnp.dot(hi, W_iu, preferred_element_type=F32)
    d_iu = jnp.dot(hu, W_iu, preferred_element_type=F32)
    s_ui = jnp.dot(hu, W_ui, preferred_element_type=F32)
    d_ui = jnp.dot(hi, W_ui, preferred_element_type=F32)
    hs_ref[0] = s_iu
    hs_ref[1] = s_ui
    als_ref[0] = _bcast16(jnp.sum(s_iu * as_iu, axis=1, keepdims=True))
    als_ref[1] = _bcast16(jnp.sum(s_ui * as_ui, axis=1, keepdims=True))
    ald_ref[0] = _bcast16(jnp.sum(d_iu * ad_iu, axis=1, keepdims=True))
    ald_ref[1] = _bcast16(jnp.sum(d_ui * ad_ui, axis=1, keepdims=True))


def _proj0_body(xu_ref, xi_ref, Wp_ref, bp_ref,
                W_iu_ref, as_iu_ref, ad_iu_ref,
                W_ui_ref, as_ui_ref, ad_ui_ref,
                hs_ref, als_ref, ald_ref):
    Wp = Wp_ref[...]
    hu = jnp.dot(xu_ref[...], Wp, preferred_element_type=F32) + bp_ref[...]
    hi = jnp.dot(xi_ref[...], Wp, preferred_element_type=F32) + bp_ref[...]
    _proj_common(hu, hi, W_iu_ref[...], as_iu_ref[...], ad_iu_ref[...],
                 W_ui_ref[...], as_ui_ref[...], ad_ui_ref[...],
                 hs_ref, als_ref, ald_ref)


def _proj1_body(acc_ref, biu_ref, bui_ref,
                W_iu_ref, as_iu_ref, ad_iu_ref,
                W_ui_ref, as_ui_ref, ad_ui_ref,
                hs_ref, als_ref, ald_ref):
    hu = _elu(acc_ref[0] + biu_ref[...])
    hi = _elu(acc_ref[1] + bui_ref[...])
    _proj_common(hu, hi, W_iu_ref[...], as_iu_ref[...], ad_iu_ref[...],
                 W_ui_ref[...], as_ui_ref[...], ad_ui_ref[...],
                 hs_ref, als_ref, ald_ref)


def _final_body(acc_ref, biu_ref, bui_ref, Wo_ref, bo_ref,
                outu_ref, hi2_ref):
    hu2 = _elu(acc_ref[0] + biu_ref[...])
    hi2 = _elu(acc_ref[1] + bui_ref[...])
    outu_ref[...] = (jnp.dot(hu2, Wo_ref[...], preferred_element_type=F32)
                     + bo_ref[...])
    hi2_ref[...] = hi2


def _row_spec(width):
    return pl.BlockSpec((BLK, width), lambda i: (i, 0))


def _st_spec(width):
    return pl.BlockSpec((2, BLK, width), lambda i: (0, i, 0))


def _full_spec(r, cdim):
    return pl.BlockSpec((r, cdim), lambda i: (0, 0))


_proj0 = pl.pallas_call(
    _proj0_body,
    grid=(GRID,),
    in_specs=[_row_spec(D), _row_spec(D), _full_spec(D, D), _full_spec(1, D),
              _full_spec(D, D), _full_spec(1, D), _full_spec(1, D),
              _full_spec(D, D), _full_spec(1, D), _full_spec(1, D)],
    out_specs=[_st_spec(D), _st_spec(16), _st_spec(16)],
    out_shape=[jax.ShapeDtypeStruct((2, N, D), F32),
               jax.ShapeDtypeStruct((2, N, 16), F32),
               jax.ShapeDtypeStruct((2, N, 16), F32)],
)

_proj1 = pl.pallas_call(
    _proj1_body,
    grid=(GRID,),
    in_specs=[_st_spec(D), _full_spec(1, D), _full_spec(1, D),
              _full_spec(D, D), _full_spec(1, D), _full_spec(1, D),
              _full_spec(D, D), _full_spec(1, D), _full_spec(1, D)],
    out_specs=[_st_spec(D), _st_spec(16), _st_spec(16)],
    out_shape=[jax.ShapeDtypeStruct((2, N, D), F32),
               jax.ShapeDtypeStruct((2, N, 16), F32),
               jax.ShapeDtypeStruct((2, N, 16), F32)],
)

_final = pl.pallas_call(
    _final_body,
    grid=(GRID,),
    in_specs=[_st_spec(D), _full_spec(1, D), _full_spec(1, D),
              _full_spec(D, 64), _full_spec(1, 64)],
    out_specs=[_row_spec(64), _row_spec(D)],
    out_shape=[jax.ShapeDtypeStruct((N, 64), F32),
               jax.ShapeDtypeStruct((N, D), F32)],
)


def kernel(x_user, x_item, edge_index_ui, edge_index_iu, Wp, bp,
           W_ui0, as_ui0, ad_ui0, b_ui0, W_iu0, as_iu0, ad_iu0, b_iu0,
           W_ui1, as_ui1, ad_ui1, b_ui1, W_iu1, as_iu1, ad_iu1, b_iu1,
           Wo, bo):
    ei_ui = edge_index_ui.astype(I32)
    ei_iu = edge_index_iu.astype(I32)
    packed = jnp.stack([
        (ei_iu[0] | (ei_iu[1] << 16)).reshape(NT, NCH, CH),
        (ei_ui[0] | (ei_ui[1] << 16)).reshape(NT, NCH, CH)])
    bp2 = bp.reshape(1, D)

    hs0, als0, ald0 = _proj0(x_user, x_item, Wp, bp2,
                             W_iu0, as_iu0, ad_iu0,
                             W_ui0, as_ui0, ad_ui0)
    acc0 = _sc_gat(hs0, als0, ald0, packed)

    hs1, als1, ald1 = _proj1(acc0, b_iu0.reshape(1, D), b_ui0.reshape(1, D),
                             W_iu1, as_iu1, ad_iu1,
                             W_ui1, as_ui1, ad_ui1)
    acc1 = _sc_gat(hs1, als1, ald1, packed)

    out_user, hi2 = _final(acc1, b_iu1.reshape(1, D), b_ui1.reshape(1, D),
                           Wo, bo.reshape(1, 64))
    return (out_user, hi2)


# speedup vs baseline: 1.1534x; 1.0038x over previous
import jax
import jax.numpy as jnp
from jax import lax
from jax.experimental import pallas as pl
from jax.experimental.pallas import tpu as pltpu
from jax.experimental.pallas import tpu_sc as plsc

N = 10000
E = 160000
D = 128
NT = 16
EPT = E // NT
CH = 80
NCH = EPT // CH
STR = 624
F32 = jnp.float32
I32 = jnp.int32


def _sc_body(hs, als, ald, packed, acc,
             packed_v, sidx0, sidx1, didx0, didx1,
             rows0, rows1, alc0, alc1, adc0, adc1, px0, px1, denr0, denr1,
             gsem0, gsem1, ssem0, ssem1,
             accs, dens):
    c = lax.axis_index("c")
    s = lax.axis_index("s")
    sidx = [sidx0, sidx1]
    didx = [didx0, didx1]
    rows = [rows0, rows1]
    alc = [alc0, alc1]
    adc = [adc0, adc1]
    px = [px0, px1]
    denr = [denr0, denr1]
    gsem = [gsem0, gsem1]
    ssem = [ssem0, ssem1]

    def zrow_step(r, carry):
        for j in range(D // 16):
            rows0[r, pl.ds(j * 16, 16)] = jnp.zeros((16,), F32)
        denr0[r, :] = jnp.zeros((16,), F32)
        return carry
    lax.fori_loop(0, CH, zrow_step, 0, unroll=True)

    for k in range(7):
        base = s * STR + k * CH
        pltpu.sync_copy(rows0, accs.at[pl.ds(base, CH), :])
        pltpu.sync_copy(denr0, dens.at[pl.ds(base, CH), :])
    pltpu.sync_copy(rows0.at[pl.ds(0, 64), :],
                    accs.at[pl.ds(s * STR + 560, 64), :])
    pltpu.sync_copy(denr0.at[pl.ds(0, 64), :],
                    dens.at[pl.ds(s * STR + 560, 64), :])

    @pl.when(s == NT - 1)
    def _():
        pltpu.sync_copy(rows0.at[pl.ds(0, 16), :],
                        accs.at[pl.ds(NT * STR, 16), :])
        pltpu.sync_copy(denr0.at[pl.ds(0, 16), :],
                        dens.at[pl.ds(NT * STR, 16), :])

    pltpu.sync_copy(packed.at[c, s], packed_v)
    plsc.subcore_barrier()

    rowid = jnp.arange(16, dtype=I32)
    zero16 = jnp.zeros((16,), I32)

    def issue(rr, b):
        @plsc.parallel_loop(0, CH // 16)
        def _(j):
            p = packed_v[rr, pl.ds(j * 16, 16)]
            sidx[b][pl.ds(j * 16, 16)] = p & 0xFFFF
            didx[b][pl.ds(j * 16, 16)] = p >> 16
        pltpu.async_copy(als.at[c].at[sidx[b]], alc[b], gsem[b])
        pltpu.async_copy(ald.at[c].at[didx[b]], adc[b], gsem[b])
        pltpu.async_copy(hs.at[c].at[sidx[b]], rows[b], gsem[b])

    def drain_scatter(b):
        pltpu.make_async_copy(rows[b], accs.at[didx[b]], ssem[b]).wait()
        pltpu.make_async_copy(denr[b], dens.at[didx[b]], ssem[b]).wait()

    def consume(b, async_scatter):
        pltpu.make_async_copy(als.at[c].at[sidx[b]], alc[b], gsem[b]).wait()
        pltpu.make_async_copy(ald.at[c].at[didx[b]], adc[b], gsem[b]).wait()
        pltpu.make_async_copy(hs.at[c].at[sidx[b]], rows[b], gsem[b]).wait()
        for j in range(CH // 16):
            ridx = rowid + (j * 16)
            a = (plsc.load_gather(alc[b], [ridx, zero16])
                 + plsc.load_gather(adc[b], [ridx, zero16]))
            a = jnp.where(a > 0, a, 0.2 * a)
            px[b][pl.ds(j * 16, 16)] = jnp.exp(a)

        @plsc.parallel_loop(0, CH, unroll=4)
        def _(e):
            w = plsc.load_gather(px[b], [jnp.full((16,), e, I32)])
            for j in range(D // 16):
                rows[b][e, pl.ds(j * 16, 16)] = (
                    rows[b][e, pl.ds(j * 16, 16)] * w)
            denr[b][e, :] = w

        if async_scatter:
            pltpu.async_copy(rows[b], accs.at[didx[b]], ssem[b], add=True)
            pltpu.async_copy(denr[b], dens.at[didx[b]], ssem[b], add=True)
        else:
            pltpu.sync_copy(rows[b], accs.at[didx[b]], add=True)
            pltpu.sync_copy(denr[b], dens.at[didx[b]], add=True)

    issue(0, 0)

    def step2(i, carry):
        r = i * 2
        @pl.when(i > 0)
        def _():
            drain_scatter(1)
        issue(r + 1, 1)
        consume(0, True)
        drain_scatter(0)

        @pl.when(r + 2 < NCH)
        def _():
            issue(r + 2, 0)
        consume(1, True)
        return carry
    lax.fori_loop(0, NCH // 2, step2, 0)
    drain_scatter(1)
    consume(0, False)
    plsc.subcore_barrier()

    def fl_nrows(k):
        return 64 if k == 7 else CH

    def fl_base(k):
        return s * STR + k * CH

    def fl_in(k, b):
        nr = fl_nrows(k)
        pltpu.async_copy(accs.at[pl.ds(fl_base(k), nr), :],
                         rows[b].at[pl.ds(0, nr), :], gsem[b])
        pltpu.async_copy(dens.at[pl.ds(fl_base(k), nr), :],
                         denr[b].at[pl.ds(0, nr), :], gsem[b])

    def fl_in_wait(k, b):
        nr = fl_nrows(k)
        pltpu.make_async_copy(accs.at[pl.ds(fl_base(k), nr), :],
                              rows[b].at[pl.ds(0, nr), :], gsem[b]).wait()
        pltpu.make_async_copy(dens.at[pl.ds(fl_base(k), nr), :],
                              denr[b].at[pl.ds(0, nr), :], gsem[b]).wait()

    def fl_norm(k, b):
        @plsc.parallel_loop(0, fl_nrows(k))
        def _(rr):
            inv = 1.0 / (denr[b][rr, :] + 1e-16)
            for j in range(D // 16):
                rows[b][rr, pl.ds(j * 16, 16)] = (
                    rows[b][rr, pl.ds(j * 16, 16)] * inv)

    def fl_out(k, b):
        nr = fl_nrows(k)
        pltpu.async_copy(rows[b].at[pl.ds(0, nr), :],
                         acc.at[c].at[pl.ds(fl_base(k), nr), :], ssem[b])

    def fl_out_wait(k, b):
        nr = fl_nrows(k)
        pltpu.make_async_copy(rows[b].at[pl.ds(0, nr), :],
                              acc.at[c].at[pl.ds(fl_base(k), nr), :],
                              ssem[b]).wait()

    fl_in(0, 0)
    for k in range(8):
        b = k & 1
        if k >= 1:
            fl_out_wait(k - 1, 1 - b)
        if k + 1 < 8:
            fl_in(k + 1, 1 - b)
        fl_in_wait(k, b)
        fl_norm(k, b)
        fl_out(k, b)
    fl_out_wait(7, 1)

    @pl.when(s == NT - 1)
    def _():
        pltpu.sync_copy(accs.at[pl.ds(NT * STR, 16), :],
                        rows0.at[pl.ds(0, 16), :])
        pltpu.sync_copy(dens.at[pl.ds(NT * STR, 16), :],
                        denr0.at[pl.ds(0, 16), :])

        @plsc.parallel_loop(0, 16)
        def _(rr):
            inv = 1.0 / (denr0[rr, :] + 1e-16)
            for j in range(D // 16):
                rows0[rr, pl.ds(j * 16, 16)] = (
                    rows0[rr, pl.ds(j * 16, 16)] * inv)
        pltpu.sync_copy(rows0.at[pl.ds(0, 16), :],
                        acc.at[c].at[pl.ds(NT * STR, 16), :])


_sc_gat = pl.kernel(
    _sc_body,
    out_type=jax.ShapeDtypeStruct((2, N, D), F32),
    mesh=plsc.VectorSubcoreMesh(core_axis_name="c", subcore_axis_name="s"),
    compiler_params=pltpu.CompilerParams(needs_layout_passes=False,
                                         use_tc_tiling_on_sc=False),
    scratch_types=[
        pltpu.VMEM((NCH, CH), I32),
        pltpu.VMEM((CH,), I32),
        pltpu.VMEM((CH,), I32),
        pltpu.VMEM((CH,), I32),
        pltpu.VMEM((CH,), I32),
        pltpu.VMEM((CH, D), F32),
        pltpu.VMEM((CH, D), F32),
        pltpu.VMEM((CH, 16), F32),
        pltpu.VMEM((CH, 16), F32),
        pltpu.VMEM((CH, 16), F32),
        pltpu.VMEM((CH, 16), F32),
        pltpu.VMEM((CH,), F32),
        pltpu.VMEM((CH,), F32),
        pltpu.VMEM((CH, 16), F32),
        pltpu.VMEM((CH, 16), F32),
        pltpu.SemaphoreType.DMA,
        pltpu.SemaphoreType.DMA,
        pltpu.SemaphoreType.DMA,
        pltpu.SemaphoreType.DMA,
        pltpu.VMEM_SHARED((N, D), F32),
        pltpu.VMEM_SHARED((N, 16), F32),
    ],
)


BLK = 1000
GRID = N // BLK


def _elu(x):
    return jnp.where(x > 0, x, jnp.exp(jnp.minimum(x, 0.0)) - 1.0)


def _bcast16(v):
    return jnp.broadcast_to(v, (v.shape[0], 16))


def _proj_common(hu, hi, W_iu, as_iu, ad_iu, W_ui, as_ui, ad_ui,
                 hs_ref, als_ref, ald_ref):
    s_iu = jnp.dot(hi, W_iu, preferred_element_type=F32)
    d_iu = jnp.dot(hu, W_iu, preferred_element_type=F32)
    s_ui = jnp.dot(hu, W_ui, preferred_element_type=F32)
    d_ui = jnp.dot(hi, W_ui, preferred_element_type=F32)
    hs_ref[0] = s_iu
    hs_ref[1] = s_ui
    als_ref[0] = _bcast16(jnp.sum(s_iu * as_iu, axis=1, keepdims=True))
    als_ref[1] = _bcast16(jnp.sum(s_ui * as_ui, axis=1, keepdims=True))
    ald_ref[0] = _bcast16(jnp.sum(d_iu * ad_iu, axis=1, keepdims=True))
    ald_ref[1] = _bcast16(jnp.sum(d_ui * ad_ui, axis=1, keepdims=True))


def _proj0_body(xu_ref, xi_ref, Wp_ref, bp_ref,
                W_iu_ref, as_iu_ref, ad_iu_ref,
                W_ui_ref, as_ui_ref, ad_ui_ref,
                hs_ref, als_ref, ald_ref):
    Wp = Wp_ref[...]
    hu = jnp.dot(xu_ref[...], Wp, preferred_element_type=F32) + bp_ref[...]
    hi = jnp.dot(xi_ref[...], Wp, preferred_element_type=F32) + bp_ref[...]
    _proj_common(hu, hi, W_iu_ref[...], as_iu_ref[...], ad_iu_ref[...],
                 W_ui_ref[...], as_ui_ref[...], ad_ui_ref[...],
                 hs_ref, als_ref, ald_ref)


def _proj1_body(acc_ref, biu_ref, bui_ref,
                W_iu_ref, as_iu_ref, ad_iu_ref,
                W_ui_ref, as_ui_ref, ad_ui_ref,
                hs_ref, als_ref, ald_ref):
    hu = _elu(acc_ref[0] + biu_ref[...])
    hi = _elu(acc_ref[1] + bui_ref[...])
    _proj_common(hu, hi, W_iu_ref[...], as_iu_ref[...], ad_iu_ref[...],
                 W_ui_ref[...], as_ui_ref[...], ad_ui_ref[...],
                 hs_ref, als_ref, ald_ref)


def _final_body(acc_ref, biu_ref, bui_ref, Wo_ref, bo_ref,
                outu_ref, hi2_ref):
    hu2 = _elu(acc_ref[0] + biu_ref[...])
    hi2 = _elu(acc_ref[1] + bui_ref[...])
    outu_ref[...] = (jnp.dot(hu2, Wo_ref[...], preferred_element_type=F32)
                     + bo_ref[...])
    hi2_ref[...] = hi2


def _row_spec(width):
    return pl.BlockSpec((BLK, width), lambda i: (i, 0))


def _st_spec(width):
    return pl.BlockSpec((2, BLK, width), lambda i: (0, i, 0))


def _full_spec(r, cdim):
    return pl.BlockSpec((r, cdim), lambda i: (0, 0))


_proj0 = pl.pallas_call(
    _proj0_body,
    grid=(GRID,),
    in_specs=[_row_spec(D), _row_spec(D), _full_spec(D, D), _full_spec(1, D),
              _full_spec(D, D), _full_spec(1, D), _full_spec(1, D),
              _full_spec(D, D), _full_spec(1, D), _full_spec(1, D)],
    out_specs=[_st_spec(D), _st_spec(16), _st_spec(16)],
    out_shape=[jax.ShapeDtypeStruct((2, N, D), F32),
               jax.ShapeDtypeStruct((2, N, 16), F32),
               jax.ShapeDtypeStruct((2, N, 16), F32)],
)

_proj1 = pl.pallas_call(
    _proj1_body,
    grid=(GRID,),
    in_specs=[_st_spec(D), _full_spec(1, D), _full_spec(1, D),
              _full_spec(D, D), _full_spec(1, D), _full_spec(1, D),
              _full_spec(D, D), _full_spec(1, D), _full_spec(1, D)],
    out_specs=[_st_spec(D), _st_spec(16), _st_spec(16)],
    out_shape=[jax.ShapeDtypeStruct((2, N, D), F32),
               jax.ShapeDtypeStruct((2, N, 16), F32),
               jax.ShapeDtypeStruct((2, N, 16), F32)],
)

_final = pl.pallas_call(
    _final_body,
    grid=(GRID,),
    in_specs=[_st_spec(D), _full_spec(1, D), _full_spec(1, D),
              _full_spec(D, 64), _full_spec(1, 64)],
    out_specs=[_row_spec(64), _row_spec(D)],
    out_shape=[jax.ShapeDtypeStruct((N, 64), F32),
               jax.ShapeDtypeStruct((N, D), F32)],
)


def kernel(x_user, x_item, edge_index_ui, edge_index_iu, Wp, bp,
           W_ui0, as_ui0, ad_ui0, b_ui0, W_iu0, as_iu0, ad_iu0, b_iu0,
           W_ui1, as_ui1, ad_ui1, b_ui1, W_iu1, as_iu1, ad_iu1, b_iu1,
           Wo, bo):
    ei_ui = edge_index_ui.astype(I32)
    ei_iu = edge_index_iu.astype(I32)
    packed = jnp.stack([
        (ei_iu[0] | (ei_iu[1] << 16)).reshape(NT, NCH, CH),
        (ei_ui[0] | (ei_ui[1] << 16)).reshape(NT, NCH, CH)])
    bp2 = bp.reshape(1, D)

    hs0, als0, ald0 = _proj0(x_user, x_item, Wp, bp2,
                             W_iu0, as_iu0, ad_iu0,
                             W_ui0, as_ui0, ad_ui0)
    acc0 = _sc_gat(hs0, als0, ald0, packed)

    hs1, als1, ald1 = _proj1(acc0, b_iu0.reshape(1, D), b_ui0.reshape(1, D),
                             W_iu1, as_iu1, ad_iu1,
                             W_ui1, as_ui1, ad_ui1)
    acc1 = _sc_gat(hs1, als1, ald1, packed)

    out_user, hi2 = _final(acc1, b_iu1.reshape(1, D), b_ui1.reshape(1, D),
                           Wo, bo.reshape(1, 64))
    return (out_user, hi2)
